# baseline (speedup 1.0000x reference)
import jax
import jax.numpy as jnp
from jax.experimental import pallas as pl


def _fused_kernel(x_ref, w_ref, t_ref, q_ref, o_ref):
    feats = jnp.dot(x_ref[...], w_ref[...], preferred_element_type=jnp.float32)
    f = (feats > t_ref[...]).astype(jnp.bfloat16)
    o_ref[...] = jax.lax.dot_general(
        f, q_ref[...], (((1,), (1,)), ((), ())),
        preferred_element_type=jnp.float32)


def kernel(x, W, b, predicate_matrix):
    bsz, d_in = x.shape
    nf = W.shape[1]
    nc = predicate_matrix.shape[0]
    bm = 512
    t = (0.5 - b).reshape(1, nf)
    q = (predicate_matrix - 1.0).astype(jnp.bfloat16)
    return pl.pallas_call(
        _fused_kernel,
        grid=(bsz // bm,),
        in_specs=[
            pl.BlockSpec((bm, d_in), lambda i: (i, 0)),
            pl.BlockSpec((d_in, nf), lambda i: (0, 0)),
            pl.BlockSpec((1, nf), lambda i: (0, 0)),
            pl.BlockSpec((nc, nf), lambda i: (0, 0)),
        ],
        out_specs=pl.BlockSpec((bm, nc), lambda i: (i, 0)),
        out_shape=jax.ShapeDtypeStruct((bsz, nc), jnp.float32),
    )(x, W, t, q)


# speedup vs baseline: 1.2337x; 1.2337x over previous
import jax
import jax.numpy as jnp
from jax.experimental import pallas as pl


def _fused_kernel(x_ref, w_ref, b_ref, p_ref, o_ref):
    feats = jnp.dot(x_ref[...], w_ref[...], preferred_element_type=jnp.float32)
    f = (feats > (0.5 - b_ref[...])).astype(jnp.bfloat16)
    q = (p_ref[...] - 1.0).astype(jnp.bfloat16)
    o_ref[...] = jax.lax.dot_general(
        f, q, (((1,), (1,)), ((), ())),
        preferred_element_type=jnp.float32)


def kernel(x, W, b, predicate_matrix):
    bsz, d_in = x.shape
    nf = W.shape[1]
    nc = predicate_matrix.shape[0]
    bm = 512
    b2 = b.reshape(1, nf)
    return pl.pallas_call(
        _fused_kernel,
        grid=(bsz // bm,),
        in_specs=[
            pl.BlockSpec((bm, d_in), lambda i: (i, 0)),
            pl.BlockSpec((d_in, nf), lambda i: (0, 0)),
            pl.BlockSpec((1, nf), lambda i: (0, 0)),
            pl.BlockSpec((nc, nf), lambda i: (0, 0)),
        ],
        out_specs=pl.BlockSpec((bm, nc), lambda i: (i, 0)),
        out_shape=jax.ShapeDtypeStruct((bsz, nc), jnp.float32),
    )(x, W, b2, predicate_matrix)
